# Initial kernel scaffold; baseline (speedup 1.0000x reference)
import jax
import jax.numpy as jnp
from jax.experimental import pallas as pl


def kernel(x, edge_index, edge_attr, u, batch, W1, b1, gamma, beta, W2, b2):
    raise NotImplementedError("write your pallas kernel here")


# speedup vs baseline: 3.8689x; 3.8689x over previous
import functools

import jax
import jax.numpy as jnp
from jax import lax
from jax.experimental import pallas as pl
from jax.experimental.pallas import tpu as pltpu
from jax.experimental.pallas import tpu_sc as plsc

N_NODES = 10000
D_FEAT = 128
NUM_GRAPHS = 256
GLOBAL_DIM = 64
HIDDEN_DIM = 256

CHUNK = 80
N_CHUNKS = N_NODES // CHUNK
N_WORKERS = 32
CHUNKS_PER_W = (N_CHUNKS + N_WORKERS - 1) // N_WORKERS
CNT_W = 128


def _sc_segment_sums(x, batch_i32):
    mesh = plsc.VectorSubcoreMesh(core_axis_name="c", subcore_axis_name="s")

    @functools.partial(
        pl.kernel,
        out_type=[
            jax.ShapeDtypeStruct((2 * NUM_GRAPHS, D_FEAT), jnp.float32),
            jax.ShapeDtypeStruct((2 * NUM_GRAPHS, CNT_W), jnp.float32),
        ],
        mesh=mesh,
        scratch_types=[
            pltpu.VMEM((CHUNK,), jnp.int32),
            pltpu.VMEM((CHUNK, D_FEAT), jnp.float32),
            pltpu.VMEM((CHUNK, CNT_W), jnp.float32),
            pltpu.VMEM((16, D_FEAT), jnp.float32),
            pltpu.VMEM_SHARED((NUM_GRAPHS, D_FEAT), jnp.float32),
            pltpu.VMEM_SHARED((NUM_GRAPHS, CNT_W), jnp.float32),
        ],
    )
    def seg(x_hbm, b_hbm, sums_out, cnts_out,
            idx_v, x_v, ones_v, z_v, sums_sh, cnts_sh):
        cid = lax.axis_index("c")
        sid = lax.axis_index("s")
        wid = cid * 16 + sid

        zero16 = jnp.zeros((16,), jnp.float32)
        one16 = jnp.ones((16,), jnp.float32)
        for i in range(16):
            for k in range(D_FEAT // 16):
                z_v[i, pl.ds(k * 16, 16)] = zero16
        for i in range(CHUNK):
            for k in range(CNT_W // 16):
                ones_v[i, pl.ds(k * 16, 16)] = one16

        pltpu.sync_copy(z_v, sums_sh.at[pl.ds(sid * 16, 16)])
        pltpu.sync_copy(z_v, cnts_sh.at[pl.ds(sid * 16, 16)])
        plsc.subcore_barrier()

        for j in range(CHUNKS_PER_W):
            t = wid + N_WORKERS * j

            @pl.when(t < N_CHUNKS)
            def _():
                base = t * CHUNK
                pltpu.sync_copy(b_hbm.at[pl.ds(base, CHUNK)], idx_v)
                pltpu.sync_copy(x_hbm.at[pl.ds(base, CHUNK)], x_v)
                pltpu.sync_copy(x_v, sums_sh.at[idx_v], add=True)
                pltpu.sync_copy(ones_v, cnts_sh.at[idx_v], add=True)

        plsc.subcore_barrier()
        row = cid * NUM_GRAPHS + sid * 16
        pltpu.sync_copy(sums_sh.at[pl.ds(sid * 16, 16)],
                        sums_out.at[pl.ds(row, 16)])
        pltpu.sync_copy(cnts_sh.at[pl.ds(sid * 16, 16)],
                        cnts_out.at[pl.ds(row, 16)])

    return seg(x, batch_i32)


def _mlp_body(sums_ref, cnts_ref, u_ref, w1_ref, b1_ref, gamma_ref,
              beta_ref, w2_ref, b2_ref, out_ref):
    s = sums_ref[...]
    total = s[:NUM_GRAPHS] + s[NUM_GRAPHS:]
    c = cnts_ref[...]
    cnt = c[:NUM_GRAPHS, 0:1] + c[NUM_GRAPHS:, 0:1]
    mean = total / jnp.maximum(cnt, 1.0)

    w1 = w1_ref[...]
    h = (jnp.dot(u_ref[...], w1[:GLOBAL_DIM], preferred_element_type=jnp.float32)
         + jnp.dot(mean, w1[GLOBAL_DIM:], preferred_element_type=jnp.float32)
         + b1_ref[...])
    mu = jnp.mean(h, axis=0, keepdims=True)
    var = jnp.mean((h - mu) ** 2, axis=0, keepdims=True)
    hn = (h - mu) * lax.rsqrt(var + 1e-5) * gamma_ref[...] + beta_ref[...]
    hn = jnp.maximum(hn, 0.0)
    out_ref[...] = (jnp.dot(hn, w2_ref[...], preferred_element_type=jnp.float32)
                    + b2_ref[...])


def kernel(x, edge_index, edge_attr, u, batch, W1, b1, gamma, beta, W2, b2):
    del edge_index, edge_attr
    batch_i32 = batch.astype(jnp.int32)
    sums, cnts = _sc_segment_sums(x, batch_i32)
    out = pl.pallas_call(
        _mlp_body,
        out_shape=jax.ShapeDtypeStruct((NUM_GRAPHS, GLOBAL_DIM), jnp.float32),
    )(sums, cnts, u, W1,
      b1.reshape(1, HIDDEN_DIM), gamma.reshape(1, HIDDEN_DIM),
      beta.reshape(1, HIDDEN_DIM), W2, b2.reshape(1, GLOBAL_DIM))
    return out


# speedup vs baseline: 4.1452x; 1.0714x over previous
import functools

import jax
import jax.numpy as jnp
from jax import lax
from jax.experimental import pallas as pl
from jax.experimental.pallas import tpu as pltpu
from jax.experimental.pallas import tpu_sc as plsc

N_NODES = 10000
D_FEAT = 128
NUM_GRAPHS = 256
GLOBAL_DIM = 64
HIDDEN_DIM = 256

CHUNK = 80
N_CHUNKS = N_NODES // CHUNK
N_WORKERS = 32
FULL_WORKERS = N_CHUNKS - 3 * N_WORKERS


def _sc_segment_sums(x, batch_i32, zeros_c, ones_c):
    mesh = plsc.VectorSubcoreMesh(core_axis_name="c", subcore_axis_name="s")

    @functools.partial(
        pl.kernel,
        out_type=[
            jax.ShapeDtypeStruct((2 * NUM_GRAPHS, D_FEAT), jnp.float32),
            jax.ShapeDtypeStruct((2 * NUM_GRAPHS, D_FEAT), jnp.float32),
        ],
        mesh=mesh,
        scratch_types=(
            [pltpu.VMEM((CHUNK,), jnp.int32) for _ in range(4)]
            + [pltpu.VMEM((CHUNK, D_FEAT), jnp.float32) for _ in range(4)]
            + [pltpu.VMEM((CHUNK, D_FEAT), jnp.float32)]
            + [pltpu.SemaphoreType.DMA for _ in range(9)]
            + [pltpu.VMEM_SHARED((NUM_GRAPHS, D_FEAT), jnp.float32),
               pltpu.VMEM_SHARED((NUM_GRAPHS, D_FEAT), jnp.float32)]
        ),
    )
    def seg(x_hbm, b_hbm, z_hbm, o_hbm, sums_out, cnts_out,
            i0, i1, i2, i3, v0, v1, v2, v3, ones_v,
            si0, si1, si2, si3, sx0, sx1, sx2, sx3, so,
            sums_sh, cnts_sh):
        cid = lax.axis_index("c")
        sid = lax.axis_index("s")
        wid = cid * 16 + sid
        idx_v = [i0, i1, i2, i3]
        x_v = [v0, v1, v2, v3]
        sem_i = [si0, si1, si2, si3]
        sem_x = [sx0, sx1, sx2, sx3]

        def start(j):
            base = (wid + N_WORKERS * j) * CHUNK
            pltpu.async_copy(b_hbm.at[pl.ds(base, CHUNK)], idx_v[j], sem_i[j])
            pltpu.async_copy(x_hbm.at[pl.ds(base, CHUNK)], x_v[j], sem_x[j])

        start(0)
        start(1)
        start(2)

        @pl.when(wid < FULL_WORKERS)
        def _():
            start(3)

        pltpu.async_copy(o_hbm, ones_v, so)
        pltpu.sync_copy(z_hbm, sums_sh.at[pl.ds(sid * 16, 16)])
        pltpu.sync_copy(z_hbm, cnts_sh.at[pl.ds(sid * 16, 16)])
        pltpu.make_async_copy(o_hbm, ones_v, so).wait()
        plsc.subcore_barrier()

        def finish(j):
            base = (wid + N_WORKERS * j) * CHUNK
            pltpu.make_async_copy(
                b_hbm.at[pl.ds(base, CHUNK)], idx_v[j], sem_i[j]).wait()
            pltpu.make_async_copy(
                x_hbm.at[pl.ds(base, CHUNK)], x_v[j], sem_x[j]).wait()
            pltpu.sync_copy(x_v[j], sums_sh.at[idx_v[j]], add=True)
            pltpu.sync_copy(ones_v, cnts_sh.at[idx_v[j]], add=True)

        finish(0)
        finish(1)
        finish(2)

        @pl.when(wid < FULL_WORKERS)
        def _():
            finish(3)

        plsc.subcore_barrier()
        row = cid * NUM_GRAPHS + sid * 16
        pltpu.sync_copy(sums_sh.at[pl.ds(sid * 16, 16)],
                        sums_out.at[pl.ds(row, 16)])
        pltpu.sync_copy(cnts_sh.at[pl.ds(sid * 16, 16)],
                        cnts_out.at[pl.ds(row, 16)])

    return seg(x, batch_i32, zeros_c, ones_c)


def _mlp_body(sums_ref, cnts_ref, u_ref, w1_ref, b1_ref, gamma_ref,
              beta_ref, w2_ref, b2_ref, out_ref):
    s = sums_ref[...]
    total = s[:NUM_GRAPHS] + s[NUM_GRAPHS:]
    c = cnts_ref[...]
    cnt = c[:NUM_GRAPHS, 0:1] + c[NUM_GRAPHS:, 0:1]
    mean = total / jnp.maximum(cnt, 1.0)

    w1 = w1_ref[...]
    h = (jnp.dot(u_ref[...], w1[:GLOBAL_DIM], preferred_element_type=jnp.float32)
         + jnp.dot(mean, w1[GLOBAL_DIM:], preferred_element_type=jnp.float32)
         + b1_ref[...])
    mu = jnp.mean(h, axis=0, keepdims=True)
    var = jnp.mean((h - mu) ** 2, axis=0, keepdims=True)
    hn = (h - mu) * lax.rsqrt(var + 1e-5) * gamma_ref[...] + beta_ref[...]
    hn = jnp.maximum(hn, 0.0)
    out_ref[...] = (jnp.dot(hn, w2_ref[...], preferred_element_type=jnp.float32)
                    + b2_ref[...])


def kernel(x, edge_index, edge_attr, u, batch, W1, b1, gamma, beta, W2, b2):
    del edge_index, edge_attr
    batch_i32 = batch.astype(jnp.int32)
    zeros_c = jnp.zeros((16, D_FEAT), jnp.float32)
    ones_c = jnp.ones((CHUNK, D_FEAT), jnp.float32)
    sums, cnts = _sc_segment_sums(x, batch_i32, zeros_c, ones_c)
    out = pl.pallas_call(
        _mlp_body,
        out_shape=jax.ShapeDtypeStruct((NUM_GRAPHS, GLOBAL_DIM), jnp.float32),
    )(sums, cnts, u, W1,
      b1.reshape(1, HIDDEN_DIM), gamma.reshape(1, HIDDEN_DIM),
      beta.reshape(1, HIDDEN_DIM), W2, b2.reshape(1, GLOBAL_DIM))
    return out


# speedup vs baseline: 4.1456x; 1.0001x over previous
import functools

import jax
import jax.numpy as jnp
import numpy as np
from jax import lax
from jax.experimental import pallas as pl
from jax.experimental.pallas import tpu as pltpu
from jax.experimental.pallas import tpu_sc as plsc

N_NODES = 10000
D_FEAT = 128
NUM_GRAPHS = 256
GLOBAL_DIM = 64
HIDDEN_DIM = 256

CHUNK = 80
N_CHUNKS = N_NODES // CHUNK
N_WORKERS = 32
FULL_WORKERS = N_CHUNKS - 3 * N_WORKERS


def _sc_segment_sums(x, batch_i32, zeros_c, ones_c):
    mesh = plsc.VectorSubcoreMesh(core_axis_name="c", subcore_axis_name="s")

    @functools.partial(
        pl.kernel,
        out_type=[
            jax.ShapeDtypeStruct((2 * NUM_GRAPHS, D_FEAT), jnp.float32),
            jax.ShapeDtypeStruct((2 * NUM_GRAPHS, D_FEAT), jnp.float32),
        ],
        mesh=mesh,
        scratch_types=(
            [pltpu.VMEM((CHUNK,), jnp.int32) for _ in range(4)]
            + [pltpu.VMEM((CHUNK, D_FEAT), jnp.float32) for _ in range(4)]
            + [pltpu.VMEM((CHUNK, D_FEAT), jnp.float32)]
            + [pltpu.SemaphoreType.DMA for _ in range(9)]
            + [pltpu.VMEM_SHARED((NUM_GRAPHS, D_FEAT), jnp.float32),
               pltpu.VMEM_SHARED((NUM_GRAPHS, D_FEAT), jnp.float32)]
        ),
    )
    def seg(x_hbm, b_hbm, z_hbm, o_hbm, sums_out, cnts_out,
            i0, i1, i2, i3, v0, v1, v2, v3, ones_v,
            si0, si1, si2, si3, sx0, sx1, sx2, sx3, so,
            sums_sh, cnts_sh):
        cid = lax.axis_index("c")
        sid = lax.axis_index("s")
        wid = cid * 16 + sid
        idx_v = [i0, i1, i2, i3]
        x_v = [v0, v1, v2, v3]
        sem_i = [si0, si1, si2, si3]
        sem_x = [sx0, sx1, sx2, sx3]

        def start(j):
            base = (wid + N_WORKERS * j) * CHUNK
            pltpu.async_copy(b_hbm.at[pl.ds(base, CHUNK)], idx_v[j], sem_i[j])
            pltpu.async_copy(x_hbm.at[pl.ds(base, CHUNK)], x_v[j], sem_x[j])

        start(0)
        start(1)
        start(2)

        @pl.when(wid < FULL_WORKERS)
        def _():
            start(3)

        pltpu.async_copy(o_hbm, ones_v, so)
        pltpu.sync_copy(z_hbm, sums_sh.at[pl.ds(sid * 16, 16)])
        pltpu.sync_copy(z_hbm, cnts_sh.at[pl.ds(sid * 16, 16)])
        pltpu.make_async_copy(o_hbm, ones_v, so).wait()
        plsc.subcore_barrier()

        def fire(j):
            base = (wid + N_WORKERS * j) * CHUNK
            pltpu.make_async_copy(
                b_hbm.at[pl.ds(base, CHUNK)], idx_v[j], sem_i[j]).wait()
            pltpu.make_async_copy(
                x_hbm.at[pl.ds(base, CHUNK)], x_v[j], sem_x[j]).wait()
            pltpu.async_copy(x_v[j], sums_sh.at[idx_v[j]], sem_x[j], add=True)
            pltpu.async_copy(ones_v, cnts_sh.at[idx_v[j]], sem_i[j], add=True)

        def drain(j):
            pltpu.make_async_copy(
                x_v[j], sums_sh.at[idx_v[j]], sem_x[j]).wait()
            pltpu.make_async_copy(
                ones_v, cnts_sh.at[idx_v[j]], sem_i[j]).wait()

        fire(0)
        fire(1)
        fire(2)

        @pl.when(wid < FULL_WORKERS)
        def _():
            fire(3)

        drain(0)
        drain(1)
        drain(2)

        @pl.when(wid < FULL_WORKERS)
        def _():
            drain(3)

        plsc.subcore_barrier()
        row = cid * NUM_GRAPHS + sid * 16
        pltpu.sync_copy(sums_sh.at[pl.ds(sid * 16, 16)],
                        sums_out.at[pl.ds(row, 16)])
        pltpu.sync_copy(cnts_sh.at[pl.ds(sid * 16, 16)],
                        cnts_out.at[pl.ds(row, 16)])

    return seg(x, batch_i32, zeros_c, ones_c)


def _mlp_body(sums_ref, cnts_ref, u_ref, w1_ref, b1_ref, gamma_ref,
              beta_ref, w2_ref, b2_ref, out_ref):
    s = sums_ref[...]
    total = s[:NUM_GRAPHS] + s[NUM_GRAPHS:]
    c = cnts_ref[...]
    cnt = c[:NUM_GRAPHS, 0:1] + c[NUM_GRAPHS:, 0:1]
    mean = total / jnp.maximum(cnt, 1.0)

    w1 = w1_ref[...]
    h = (jnp.dot(u_ref[...], w1[:GLOBAL_DIM], preferred_element_type=jnp.float32)
         + jnp.dot(mean, w1[GLOBAL_DIM:], preferred_element_type=jnp.float32)
         + b1_ref[...])
    mu = jnp.mean(h, axis=0, keepdims=True)
    var = jnp.mean((h - mu) ** 2, axis=0, keepdims=True)
    hn = (h - mu) * lax.rsqrt(var + 1e-5) * gamma_ref[...] + beta_ref[...]
    hn = jnp.maximum(hn, 0.0)
    out_ref[...] = (jnp.dot(hn, w2_ref[...], preferred_element_type=jnp.float32)
                    + b2_ref[...])


def kernel(x, edge_index, edge_attr, u, batch, W1, b1, gamma, beta, W2, b2):
    del edge_index, edge_attr
    batch_i32 = batch.astype(jnp.int32)
    zeros_c = np.zeros((16, D_FEAT), np.float32)
    ones_c = np.ones((CHUNK, D_FEAT), np.float32)
    sums, cnts = _sc_segment_sums(x, batch_i32, zeros_c, ones_c)
    out = pl.pallas_call(
        _mlp_body,
        out_shape=jax.ShapeDtypeStruct((NUM_GRAPHS, GLOBAL_DIM), jnp.float32),
    )(sums, cnts, u, W1,
      b1.reshape(1, HIDDEN_DIM), gamma.reshape(1, HIDDEN_DIM),
      beta.reshape(1, HIDDEN_DIM), W2, b2.reshape(1, GLOBAL_DIM))
    return out
